# baseline (speedup 1.0000x reference)
import functools

import jax
import jax.numpy as jnp
from jax import lax
from jax.experimental import pallas as pl
from jax.experimental.pallas import tpu as pltpu
from jax.experimental.pallas import tpu_sc as plsc

N_NODES = 10000
N_PAD = 10240
N_EDGES = 320000
NW = 32
K = 8
G = 10
NB = K * G
E_PAD = NW * NB * 128
D_FEAT = 128
D_HID = 16
N_CLASSES = 7
ROWS_PER_TILE = N_PAD // 16


def _sc_mesh():
    return plsc.VectorSubcoreMesh(
        core_axis_name="c", subcore_axis_name="s", num_cores=2, num_subcores=16
    )


def _deg_body(dstg, degp, dst_v, ones_v, zbuf_v, deg_sh, sem):
    c = lax.axis_index("c")
    s = lax.axis_index("s")
    wid = c * 16 + s

    def fill_ones(i, carry):
        ones_v[pl.ds(i * 16, 16)] = jnp.ones((16,), jnp.float32)
        return carry

    lax.fori_loop(0, 8, fill_ones, 0)

    def fill_zeros(i, carry):
        zbuf_v[pl.ds(i * 16, 16)] = jnp.zeros((16,), jnp.float32)
        return carry

    lax.fori_loop(0, ROWS_PER_TILE // 16, fill_zeros, 0)

    pltpu.sync_copy(dstg.at[wid], dst_v)
    pltpu.sync_copy(zbuf_v, deg_sh.at[pl.ds(s * ROWS_PER_TILE, ROWS_PER_TILE)])
    plsc.subcore_barrier()

    def body(g, carry):
        descs = [
            pltpu.async_copy(ones_v, deg_sh.at[dst_v.at[g * K + b]], sem, add=True)
            for b in range(K)
        ]
        for d in descs:
            d.wait()
        return carry

    lax.fori_loop(0, G, body, 0)
    plsc.subcore_barrier()

    pltpu.sync_copy(deg_sh.at[pl.ds(s * ROWS_PER_TILE, ROWS_PER_TILE)], zbuf_v)
    pltpu.sync_copy(zbuf_v, degp.at[c, pl.ds(s * ROWS_PER_TILE, ROWS_PER_TILE)])


def _sc_degree(dstg):
    kern = functools.partial(
        pl.kernel,
        out_type=jax.ShapeDtypeStruct((2, N_PAD), jnp.float32),
        mesh=_sc_mesh(),
        scratch_types=[
            pltpu.VMEM((NB, 128), jnp.int32),
            pltpu.VMEM((128,), jnp.float32),
            pltpu.VMEM((ROWS_PER_TILE,), jnp.float32),
            pltpu.VMEM_SHARED((N_PAD,), jnp.float32),
            pltpu.SemaphoreType.DMA,
        ],
        compiler_params=pltpu.CompilerParams(use_tc_tiling_on_sc=False),
    )(_deg_body)
    return kern(dstg)


def _agg_body(table, srcg, dstg, aggp, src_v, dst_v, rows_v, zbuf_v, acc_sh,
              table_sh, gsem, ssem):
    c = lax.axis_index("c")
    s = lax.axis_index("s")
    wid = c * 16 + s

    tile_rows = pl.ds(s * ROWS_PER_TILE, ROWS_PER_TILE)
    pltpu.sync_copy(table.at[tile_rows], zbuf_v)
    pltpu.sync_copy(zbuf_v, table_sh.at[tile_rows])

    def fill_zeros(i, carry):
        zbuf_v[i, :] = jnp.zeros((16,), jnp.float32)
        return carry

    lax.fori_loop(0, ROWS_PER_TILE, fill_zeros, 0)

    pltpu.sync_copy(srcg.at[wid], src_v)
    pltpu.sync_copy(dstg.at[wid], dst_v)
    pltpu.sync_copy(zbuf_v, acc_sh.at[tile_rows])
    plsc.subcore_barrier()

    def issue_gathers(g, setidx):
        for b in range(K):
            pltpu.async_copy(
                table_sh.at[src_v.at[g * K + b]], rows_v.at[setidx, b], gsem
            )

    issue_gathers(0, 0)

    def body(g, carry):
        cur = lax.rem(g, 2)
        nxt = lax.rem(g + 1, 2)

        @pl.when(g + 1 < G)
        def _():
            issue_gathers(g + 1, nxt)

        for b in range(K):
            pltpu.make_async_copy(
                table_sh.at[src_v.at[g * K + b]], rows_v.at[cur, b], gsem
            ).wait()
        descs = [
            pltpu.async_copy(
                rows_v.at[cur, b], acc_sh.at[dst_v.at[g * K + b]], ssem, add=True
            )
            for b in range(K)
        ]
        for d in descs:
            d.wait()
        return carry

    lax.fori_loop(0, G, body, 0)
    plsc.subcore_barrier()

    pltpu.sync_copy(acc_sh.at[tile_rows], zbuf_v)
    pltpu.sync_copy(zbuf_v, aggp.at[c, tile_rows])


def _sc_aggregate(table, srcg, dstg):
    kern = functools.partial(
        pl.kernel,
        out_type=jax.ShapeDtypeStruct((2, N_PAD, D_HID), jnp.float32),
        mesh=_sc_mesh(),
        scratch_types=[
            pltpu.VMEM((NB, 128), jnp.int32),
            pltpu.VMEM((NB, 128), jnp.int32),
            pltpu.VMEM((2, K, 128, D_HID), jnp.float32),
            pltpu.VMEM((ROWS_PER_TILE, D_HID), jnp.float32),
            pltpu.VMEM_SHARED((N_PAD, D_HID), jnp.float32),
            pltpu.VMEM_SHARED((N_PAD, D_HID), jnp.float32),
            pltpu.SemaphoreType.DMA,
            pltpu.SemaphoreType.DMA,
        ],
        compiler_params=pltpu.CompilerParams(use_tc_tiling_on_sc=False),
    )(_agg_body)
    return kern(table, srcg, dstg)


RT = N_PAD // 8
RX = N_NODES // 8


def _tc_table1_body(degp_ref, x_ref, w1e_ref, e8_ref, out_ref, dinv_ref):
    deg = degp_ref[0, :, :] + degp_ref[1, :, :] + 1.0
    dinv_n = lax.rsqrt(deg)
    dinv8 = jnp.dot(dinv_n, e8_ref[...], preferred_element_type=jnp.float32,
                    precision=lax.Precision.HIGHEST)
    dinv_ref[...] = dinv8
    h = jnp.dot(x_ref[:, 0, :], w1e_ref[0, :, :],
                preferred_element_type=jnp.float32)
    for s in range(1, 8):
        h = h + jnp.dot(x_ref[:, s, :], w1e_ref[s, :, :],
                        preferred_element_type=jnp.float32)
    out_ref[pl.ds(0, RX), :] = h * dinv8[0:RX, :]
    out_ref[pl.ds(RX, RT - RX), :] = jnp.zeros((RT - RX, 128), jnp.float32)


def _tc_table2_body(aggp_ref, t1_ref, dinv_ref, b1_ref, w2bd_ref, out_ref):
    agg = aggp_ref[0, :, :] + aggp_ref[1, :, :] + t1_ref[...]
    z1 = jnp.maximum(agg * dinv_ref[...] + b1_ref[...], 0.0)
    h2 = jnp.dot(z1, w2bd_ref[...], preferred_element_type=jnp.float32)
    out_ref[...] = h2 * dinv_ref[...]


def _tc_out_body(aggp_ref, t2_ref, dinv_ref, b2_ref, perm_ref, ones_ref,
                 out_ref):
    agg = aggp_ref[0, :, :] + aggp_ref[1, :, :] + t2_ref[...]
    z = agg * dinv_ref[...] + b2_ref[...]
    col = lax.broadcasted_iota(jnp.int32, (RT, 128), 1)
    zm = jnp.where(col % 16 < N_CLASSES, z, -1e30)
    m = zm
    for p in range(4):
        m = jnp.maximum(
            m,
            jnp.dot(m, perm_ref[p, :, :], preferred_element_type=jnp.float32,
                    precision=lax.Precision.HIGHEST),
        )
    e = jnp.exp(zm - m)
    ssum = jnp.dot(e, ones_ref[...], preferred_element_type=jnp.float32,
                   precision=lax.Precision.HIGHEST)
    out_ref[...] = z - m - jnp.log(ssum)


def kernel(x, edge_index, W1, b1, W2, b2):
    f32 = jnp.float32
    ep = jnp.pad(
        edge_index, ((0, 0), (0, E_PAD - N_EDGES)),
        constant_values=N_PAD - 1,
    )
    srcg = ep[0].reshape(NW, NB, 128)
    dstg = ep[1].reshape(NW, NB, 128)

    e8 = jnp.repeat(jnp.eye(8, dtype=f32), 16, axis=1)
    w1e = jnp.stack(
        [jnp.pad(W1, ((0, 0), (16 * s, 112 - 16 * s))) for s in range(8)]
    )
    w2p = jnp.pad(W2, ((0, 0), (0, D_HID - N_CLASSES)))
    w2bd = jnp.kron(jnp.eye(8, dtype=f32), w2p)
    b1t = jnp.tile(b1, 8).reshape(1, 128)
    b2t = jnp.tile(jnp.pad(b2, (0, D_HID - N_CLASSES)), 8).reshape(1, 128)
    perm = jnp.stack(
        [
            jnp.kron(jnp.eye(8, dtype=f32),
                     jnp.roll(jnp.eye(16, dtype=f32), sh, axis=1))
            for sh in (1, 2, 4, 8)
        ]
    )
    ones_bd = jnp.kron(jnp.eye(8, dtype=f32), jnp.ones((16, 16), f32))

    degp = _sc_degree(dstg)

    t1_8, dinv8 = pl.pallas_call(
        _tc_table1_body,
        out_shape=(
            jax.ShapeDtypeStruct((RT, 128), f32),
            jax.ShapeDtypeStruct((RT, 128), f32),
        ),
    )(degp.reshape(2, RT, 8), x.reshape(RX, 8, 128), w1e, e8)

    agg1p = _sc_aggregate(t1_8.reshape(N_PAD, D_HID), srcg, dstg)

    t2_8 = pl.pallas_call(
        _tc_table2_body,
        out_shape=jax.ShapeDtypeStruct((RT, 128), f32),
    )(agg1p.reshape(2, RT, 128), t1_8, dinv8, b1t, w2bd)

    agg2p = _sc_aggregate(t2_8.reshape(N_PAD, D_HID), srcg, dstg)

    out8 = pl.pallas_call(
        _tc_out_body,
        out_shape=jax.ShapeDtypeStruct((RT, 128), f32),
    )(agg2p.reshape(2, RT, 128), t2_8, dinv8, b2t, perm, ones_bd)

    return out8.reshape(N_PAD, D_HID)[:N_NODES, :N_CLASSES]


# speedup vs baseline: 1.0410x; 1.0410x over previous
import functools

import jax
import jax.numpy as jnp
from jax import lax
from jax.experimental import pallas as pl
from jax.experimental.pallas import tpu as pltpu
from jax.experimental.pallas import tpu_sc as plsc

N_NODES = 10000
N_PAD = 10240
N_EDGES = 320000
NW = 32
K = 8
G = 10
NB = K * G
E_PAD = NW * NB * 128
D_FEAT = 128
D_HID = 16
N_CLASSES = 7
ROWS_PER_TILE = N_PAD // 16


def _sc_mesh():
    return plsc.VectorSubcoreMesh(
        core_axis_name="c", subcore_axis_name="s", num_cores=2, num_subcores=16
    )


def _deg_body(eg, degp, dst_v, ones_v, zbuf_v, deg_sh, sem):
    c = lax.axis_index("c")
    s = lax.axis_index("s")
    wid = c * 16 + s

    def fill_ones(i, carry):
        ones_v[pl.ds(i * 16, 16)] = jnp.ones((16,), jnp.float32)
        return carry

    lax.fori_loop(0, 8, fill_ones, 0)

    def fill_zeros(i, carry):
        zbuf_v[pl.ds(i * 16, 16)] = jnp.zeros((16,), jnp.float32)
        return carry

    lax.fori_loop(0, ROWS_PER_TILE // 16, fill_zeros, 0)

    pltpu.sync_copy(eg.at[1, wid], dst_v)
    pltpu.sync_copy(zbuf_v, deg_sh.at[pl.ds(s * ROWS_PER_TILE, ROWS_PER_TILE)])
    plsc.subcore_barrier()

    def body(g, carry):
        descs = [
            pltpu.async_copy(ones_v, deg_sh.at[dst_v.at[g * K + b]], sem, add=True)
            for b in range(K)
        ]
        for d in descs:
            d.wait()
        return carry

    lax.fori_loop(0, G, body, 0)
    plsc.subcore_barrier()

    pltpu.sync_copy(deg_sh.at[pl.ds(s * ROWS_PER_TILE, ROWS_PER_TILE)], zbuf_v)
    pltpu.sync_copy(zbuf_v, degp.at[c, pl.ds(s * ROWS_PER_TILE, ROWS_PER_TILE)])


def _sc_degree(eg):
    kern = functools.partial(
        pl.kernel,
        out_type=jax.ShapeDtypeStruct((2, N_PAD), jnp.float32),
        mesh=_sc_mesh(),
        scratch_types=[
            pltpu.VMEM((NB, 128), jnp.int32),
            pltpu.VMEM((128,), jnp.float32),
            pltpu.VMEM((ROWS_PER_TILE,), jnp.float32),
            pltpu.VMEM_SHARED((N_PAD,), jnp.float32),
            pltpu.SemaphoreType.DMA,
        ],
        compiler_params=pltpu.CompilerParams(use_tc_tiling_on_sc=False),
    )(_deg_body)
    return kern(eg)


def _agg_body(table, eg, aggp, src_v, dst_v, rows_v, zbuf_v, acc_sh,
              table_sh, gsem, ssem):
    c = lax.axis_index("c")
    s = lax.axis_index("s")
    wid = c * 16 + s

    tile_rows = pl.ds(s * ROWS_PER_TILE, ROWS_PER_TILE)
    pltpu.sync_copy(table.at[tile_rows], zbuf_v)
    pltpu.sync_copy(zbuf_v, table_sh.at[tile_rows])

    def fill_zeros(i, carry):
        zbuf_v[i, :] = jnp.zeros((16,), jnp.float32)
        return carry

    lax.fori_loop(0, ROWS_PER_TILE, fill_zeros, 0)

    pltpu.sync_copy(eg.at[0, wid], src_v)
    pltpu.sync_copy(eg.at[1, wid], dst_v)
    pltpu.sync_copy(zbuf_v, acc_sh.at[tile_rows])
    plsc.subcore_barrier()

    def issue_gathers(g, setidx):
        for b in range(K):
            pltpu.async_copy(
                table_sh.at[src_v.at[g * K + b]], rows_v.at[setidx, b], gsem
            )

    issue_gathers(0, 0)

    def body(g, carry):
        cur = lax.rem(g, 2)
        nxt = lax.rem(g + 1, 2)

        @pl.when(g + 1 < G)
        def _():
            issue_gathers(g + 1, nxt)

        for b in range(K):
            pltpu.make_async_copy(
                table_sh.at[src_v.at[g * K + b]], rows_v.at[cur, b], gsem
            ).wait()
        descs = [
            pltpu.async_copy(
                rows_v.at[cur, b], acc_sh.at[dst_v.at[g * K + b]], ssem, add=True
            )
            for b in range(K)
        ]
        for d in descs:
            d.wait()
        return carry

    lax.fori_loop(0, G, body, 0)
    plsc.subcore_barrier()

    pltpu.sync_copy(acc_sh.at[tile_rows], zbuf_v)
    pltpu.sync_copy(zbuf_v, aggp.at[c, tile_rows])


def _sc_aggregate(table, eg):
    kern = functools.partial(
        pl.kernel,
        out_type=jax.ShapeDtypeStruct((2, N_PAD, D_HID), jnp.float32),
        mesh=_sc_mesh(),
        scratch_types=[
            pltpu.VMEM((NB, 128), jnp.int32),
            pltpu.VMEM((NB, 128), jnp.int32),
            pltpu.VMEM((2, K, 128, D_HID), jnp.float32),
            pltpu.VMEM((ROWS_PER_TILE, D_HID), jnp.float32),
            pltpu.VMEM_SHARED((N_PAD, D_HID), jnp.float32),
            pltpu.VMEM_SHARED((N_PAD, D_HID), jnp.float32),
            pltpu.SemaphoreType.DMA,
            pltpu.SemaphoreType.DMA,
        ],
        compiler_params=pltpu.CompilerParams(use_tc_tiling_on_sc=False),
    )(_agg_body)
    return kern(table, eg)


RT = N_PAD // 8
RX = N_NODES // 8


def _tc_table1_body(degp_ref, x_ref, w1e_ref, e8_ref, out_ref, dinv_ref):
    deg = degp_ref[0, :, :] + degp_ref[1, :, :] + 1.0
    dinv_n = lax.rsqrt(deg)
    dinv8 = jnp.dot(dinv_n, e8_ref[...], preferred_element_type=jnp.float32,
                    precision=lax.Precision.HIGHEST)
    dinv_ref[...] = dinv8
    h = jnp.dot(x_ref[...], w1e_ref[...], preferred_element_type=jnp.float32)
    out_ref[pl.ds(0, RX), :] = h * dinv8[0:RX, :]
    out_ref[pl.ds(RX, RT - RX), :] = jnp.zeros((RT - RX, 128), jnp.float32)


def _tc_table2_body(aggp_ref, t1_ref, dinv_ref, b1_ref, w2bd_ref, out_ref):
    agg = aggp_ref[0, :, :] + aggp_ref[1, :, :] + t1_ref[...]
    z1 = jnp.maximum(agg * dinv_ref[...] + b1_ref[...], 0.0)
    h2 = jnp.dot(z1, w2bd_ref[...], preferred_element_type=jnp.float32)
    out_ref[...] = h2 * dinv_ref[...]


def _tc_out_body(aggp_ref, t2_ref, dinv_ref, b2_ref, perm_ref, ones_ref,
                 out_ref):
    agg = aggp_ref[0, :, :] + aggp_ref[1, :, :] + t2_ref[...]
    z = agg * dinv_ref[...] + b2_ref[...]
    col = lax.broadcasted_iota(jnp.int32, (RT, 128), 1)
    zm = jnp.where(col % 16 < N_CLASSES, z, -1e30)
    m = zm
    for p in range(4):
        m = jnp.maximum(
            m,
            jnp.dot(m, perm_ref[p, :, :], preferred_element_type=jnp.float32,
                    precision=lax.Precision.HIGHEST),
        )
    e = jnp.exp(zm - m)
    ssum = jnp.dot(e, ones_ref[...], preferred_element_type=jnp.float32,
                   precision=lax.Precision.HIGHEST)
    out_ref[...] = z - m - jnp.log(ssum)


def kernel(x, edge_index, W1, b1, W2, b2):
    f32 = jnp.float32
    eg = jnp.pad(
        edge_index, ((0, 0), (0, E_PAD - N_EDGES)),
        constant_values=N_PAD - 1,
    ).reshape(2, NW, NB, 128)

    e8 = jnp.repeat(jnp.eye(8, dtype=f32), 16, axis=1)
    w1e = jnp.stack(
        [jnp.pad(W1, ((0, 0), (16 * s, 112 - 16 * s))) for s in range(8)]
    ).reshape(1024, 128)
    w2p = jnp.pad(W2, ((0, 0), (0, D_HID - N_CLASSES)))
    w2bd = jnp.kron(jnp.eye(8, dtype=f32), w2p)
    b1t = jnp.tile(b1, 8).reshape(1, 128)
    b2t = jnp.tile(jnp.pad(b2, (0, D_HID - N_CLASSES)), 8).reshape(1, 128)
    perm = jnp.stack(
        [
            jnp.kron(jnp.eye(8, dtype=f32),
                     jnp.roll(jnp.eye(16, dtype=f32), sh, axis=1))
            for sh in (1, 2, 4, 8)
        ]
    )
    ones_bd = jnp.kron(jnp.eye(8, dtype=f32), jnp.ones((16, 16), f32))

    degp = _sc_degree(eg)

    t1_8, dinv8 = pl.pallas_call(
        _tc_table1_body,
        out_shape=(
            jax.ShapeDtypeStruct((RT, 128), f32),
            jax.ShapeDtypeStruct((RT, 128), f32),
        ),
    )(degp.reshape(2, RT, 8), x.reshape(RX, 1024), w1e, e8)

    agg1p = _sc_aggregate(t1_8.reshape(N_PAD, D_HID), eg)

    t2_8 = pl.pallas_call(
        _tc_table2_body,
        out_shape=jax.ShapeDtypeStruct((RT, 128), f32),
    )(agg1p.reshape(2, RT, 128), t1_8, dinv8, b1t, w2bd)

    agg2p = _sc_aggregate(t2_8.reshape(N_PAD, D_HID), eg)

    out8 = pl.pallas_call(
        _tc_out_body,
        out_shape=jax.ShapeDtypeStruct((RT, 128), f32),
    )(agg2p.reshape(2, RT, 128), t2_8, dinv8, b2t, perm, ones_bd)

    return out8[:RX, :].reshape(N_NODES, D_HID)[:, :N_CLASSES]
